# baseline (speedup 1.0000x reference)
import functools

import jax
import jax.numpy as jnp
from jax import lax
from jax.experimental import pallas as pl
from jax.experimental.pallas import tpu as pltpu
from jax.experimental.pallas import tpu_sc as plsc

_F = 26
_E = 32
_V = 2600000
_B = 16384
_TOTAL = _B * _F
_W = 65536
_Q = _W // 4
_NBLK = (_V + _W - 1) // _W
_VP = _NBLK * _W
_CH = 128
_NCHUNK = _TOTAL // _CH
_CPF = _B // _CH

_SEG_F = (0, 12, 20, 26)
_RPF = _V // _F

def _seg_blocks(si):
    lo = _SEG_F[si] * _RPF // _W
    hi = min(_NBLK, (_SEG_F[si + 1] * _RPF + _W - 1) // _W)
    return lo, hi


def _untangle(tT, j0, nblk):
    def body(i0, i1, i2, i3, out_ref):
        stk = jnp.concatenate([i0[...], i1[...], i2[...], i3[...]], axis=0)
        out_ref[...] = jnp.transpose(stk)

    return pl.pallas_call(
        body,
        out_shape=jax.ShapeDtypeStruct((nblk * _Q, 128), jnp.float32),
        grid=(nblk,),
        in_specs=[
            pl.BlockSpec(
                (_E, _Q),
                (lambda j, u=u: (0, jnp.minimum(4 * (j0 + j) + u, _V // _Q))),
            )
            for u in range(4)
        ],
        out_specs=pl.BlockSpec((_Q, 128), lambda j: (j, 0)),
    )(tT, tT, tT, tT)


def _make_emb(nchunks_w, base_chunk, base_f, row_off, ngroups):
    mesh = plsc.VectorSubcoreMesh(core_axis_name="c", subcore_axis_name="s")
    K = 8
    NB = nchunks_w // K

    @functools.partial(
        pl.kernel,
        mesh=mesh,
        out_type=jax.ShapeDtypeStruct((ngroups * _B, 128), jnp.float32),
        compiler_params=pltpu.CompilerParams(use_tc_tiling_on_sc=False),
        scratch_types=[
            pltpu.VMEM((nchunks_w, _CH), jnp.int32),
            pltpu.VMEM((2, K * _CH, _E), jnp.float32),
            pltpu.SemaphoreType.DMA,
            pltpu.SemaphoreType.DMA,
        ],
    )
    def _emb(x_hbm, tab_hbm, out_hbm, idxv, rows2, semg, semw):
        info = plsc.get_sparse_core_info()
        NC = info.num_cores
        wid = lax.axis_index("s") * NC + lax.axis_index("c")
        gbase = base_chunk + wid * nchunks_w
        pltpu.sync_copy(x_hbm.at[pl.ds(gbase, nchunks_w)], idxv)

        def adjust(g, carry):
            off = lax.div(gbase + g, _CPF) * (_V // _F)
            for k in range(_CH // 16):
                sl = pl.ds(k * 16, 16)
                r = idxv[g, sl] + off
                hi = lax.bitwise_and(r, jnp.int32(~(_W - 1)))
                inb = lax.bitwise_and(r, jnp.int32(_W - 1))
                u = lax.shift_right_logical(inb, 14)
                q = lax.bitwise_and(inb, jnp.int32(_Q - 1))
                s = lax.bitwise_or(
                    lax.bitwise_or(hi, lax.shift_left(q, 2)), u)
                idxv[g, sl] = s - row_off
            return carry

        lax.fori_loop(0, nchunks_w, adjust, 0)

        def _wb_window(g):
            R = gbase + g
            fl = lax.div(R, _CPF) - base_f
            m = lax.div(fl, 4)
            v = lax.rem(fl, 4)
            b0 = lax.rem(R, _CPF) * _CH
            return out_hbm.at[pl.ds(m * _B + b0, _CH), pl.ds(32 * v, 32)]

        def batch(n, carry):
            buf = lax.rem(n, 2)

            @pl.when(n >= 2)
            def _drain_prev():
                for _ in range(K):
                    pltpu.make_async_copy(
                        rows2.at[0, pl.ds(0, _CH)], _wb_window(0), semw
                    ).wait()

            cps = [
                pltpu.async_copy(
                    tab_hbm.at[idxv.at[n * K + j]],
                    rows2.at[buf, pl.ds(j * _CH, _CH)],
                    semg,
                )
                for j in range(K)
            ]
            for c in cps:
                c.wait()
            for j in range(K):
                pltpu.async_copy(
                    rows2.at[buf, pl.ds(j * _CH, _CH)],
                    _wb_window(n * K + j),
                    semw,
                )
            return carry

        lax.fori_loop(0, NB, batch, 0)
        for _ in range(2 * K):
            pltpu.make_async_copy(
                rows2.at[0, pl.ds(0, _CH)], _wb_window(0), semw
            ).wait()

    return _emb


def _repack(gp1, gp2, gp3):
    def body(i1, i2, i3, out_ref):
        m = pl.program_id(0)
        src = jnp.where(m <= 2, i1[...],
                        jnp.where(m <= 4, i2[...], i3[...]))
        out_ref[...] = jnp.transpose(src).reshape(4, _E, _B // 2)

    return pl.pallas_call(
        body,
        out_shape=jax.ShapeDtypeStruct((_F, _E, _B), jnp.float32),
        grid=(7, 2),
        in_specs=[
            pl.BlockSpec(
                (_B // 2, 128),
                lambda m, h: (jnp.minimum(2 * m + h, 5), 0),
            ),
            pl.BlockSpec(
                (_B // 2, 128),
                lambda m, h: (jnp.clip(2 * (m - 3) + h, 0, 3), 0),
            ),
            pl.BlockSpec(
                (_B // 2, 128),
                lambda m, h: (jnp.clip(2 * (m - 5) + h, 0, 3), 0),
            ),
        ],
        out_specs=pl.BlockSpec((4, _E, _B // 2), lambda m, h: (m, 0, h)),
    )(gp1, gp2, gp3)


def kernel(x, table):
    NW = 32
    tT = table.T
    xf = x.T.reshape(_NCHUNK, _CH)

    gps = []
    for si in range(3):
        lo, hi = _seg_blocks(si)
        L = _untangle(tT, lo, hi - lo)
        Lr = L.reshape((hi - lo) * _W, _E)
        f0, f1 = _SEG_F[si], _SEG_F[si + 1]
        nchunks = (f1 - f0) * _CPF
        emb = _make_emb(nchunks // NW, f0 * _CPF, f0, lo * _W,
                        (f1 - f0 + 3) // 4)
        gps.append(emb(xf, Lr))

    Y = _repack(*gps)
    return jnp.transpose(Y, (2, 0, 1))


# speedup vs baseline: 1.0239x; 1.0239x over previous
import functools

import jax
import jax.numpy as jnp
from jax import lax
from jax.experimental import pallas as pl
from jax.experimental.pallas import tpu as pltpu
from jax.experimental.pallas import tpu_sc as plsc

_F = 26
_E = 32
_V = 2600000
_B = 16384
_TOTAL = _B * _F
_W = 65536
_Q = _W // 4
_NBLK = (_V + _W - 1) // _W
_VP = _NBLK * _W
_CH = 128
_NCHUNK = _TOTAL // _CH
_CPF = _B // _CH

_SPLIT_F = 16
_SPLIT_BLK = (_SPLIT_F * (_V // _F) + _W - 1) // _W
_SEG2_BLK = _SPLIT_F * (_V // _F) // _W
_ROW_OFF = _SEG2_BLK * _W


def _untangle(tT, j0, nblk):
    def body(i0, i1, i2, i3, out_ref):
        stk = jnp.concatenate([i0[...], i1[...], i2[...], i3[...]], axis=0)
        out_ref[...] = jnp.transpose(stk)

    return pl.pallas_call(
        body,
        out_shape=jax.ShapeDtypeStruct((nblk * _Q, 128), jnp.float32),
        grid=(nblk,),
        in_specs=[
            pl.BlockSpec(
                (_E, _Q),
                (lambda j, u=u: (0, jnp.minimum(4 * (j0 + j) + u, _V // _Q))),
            )
            for u in range(4)
        ],
        out_specs=pl.BlockSpec((_Q, 128), lambda j: (j, 0)),
    )(tT, tT, tT, tT)


def _make_emb(nchunks_w, base_chunk, base_f, row_off, ngroups):
    mesh = plsc.VectorSubcoreMesh(core_axis_name="c", subcore_axis_name="s")
    K = 8
    NB = nchunks_w // K

    @functools.partial(
        pl.kernel,
        mesh=mesh,
        out_type=jax.ShapeDtypeStruct((ngroups * _B, 128), jnp.float32),
        compiler_params=pltpu.CompilerParams(use_tc_tiling_on_sc=False),
        scratch_types=[
            pltpu.VMEM((nchunks_w, _CH), jnp.int32),
            pltpu.VMEM((2, K * _CH, _E), jnp.float32),
            pltpu.SemaphoreType.DMA,
            pltpu.SemaphoreType.DMA,
        ],
    )
    def _emb(x_hbm, tab_hbm, out_hbm, idxv, rows2, semg, semw):
        info = plsc.get_sparse_core_info()
        NC = info.num_cores
        wid = lax.axis_index("s") * NC + lax.axis_index("c")
        gbase = base_chunk + wid * nchunks_w
        pltpu.sync_copy(x_hbm.at[pl.ds(gbase, nchunks_w)], idxv)

        def adjust(g, carry):
            off = lax.div(gbase + g, _CPF) * (_V // _F)
            for k in range(_CH // 16):
                sl = pl.ds(k * 16, 16)
                r = idxv[g, sl] + off
                hi = lax.bitwise_and(r, jnp.int32(~(_W - 1)))
                inb = lax.bitwise_and(r, jnp.int32(_W - 1))
                u = lax.shift_right_logical(inb, 14)
                q = lax.bitwise_and(inb, jnp.int32(_Q - 1))
                s = lax.bitwise_or(
                    lax.bitwise_or(hi, lax.shift_left(q, 2)), u)
                idxv[g, sl] = s - row_off
            return carry

        lax.fori_loop(0, nchunks_w, adjust, 0)

        def _wb_window(g):
            R = gbase + g
            fl = lax.div(R, _CPF) - base_f
            m = lax.div(fl, 4)
            v = lax.rem(fl, 4)
            b0 = lax.rem(R, _CPF) * _CH
            return out_hbm.at[pl.ds(m * _B + b0, _CH), pl.ds(32 * v, 32)]

        def batch(n, carry):
            buf = lax.rem(n, 2)

            @pl.when(n >= 2)
            def _drain_prev():
                for _ in range(K):
                    pltpu.make_async_copy(
                        rows2.at[0, pl.ds(0, _CH)], _wb_window(0), semw
                    ).wait()

            cps = [
                pltpu.async_copy(
                    tab_hbm.at[idxv.at[n * K + j]],
                    rows2.at[buf, pl.ds(j * _CH, _CH)],
                    semg,
                )
                for j in range(K)
            ]
            for c in cps:
                c.wait()
            for j in range(K):
                pltpu.async_copy(
                    rows2.at[buf, pl.ds(j * _CH, _CH)],
                    _wb_window(n * K + j),
                    semw,
                )
            return carry

        lax.fori_loop(0, NB, batch, 0)
        for _ in range(2 * K):
            pltpu.make_async_copy(
                rows2.at[0, pl.ds(0, _CH)], _wb_window(0), semw
            ).wait()

    return _emb


def _repack(gp1, gp2):
    def body(i1, i2, out_ref):
        m = pl.program_id(0)
        src = jnp.where(m <= 3, i1[...], i2[...])
        out_ref[...] = jnp.transpose(src).reshape(4, _E, _B // 2)

    return pl.pallas_call(
        body,
        out_shape=jax.ShapeDtypeStruct((_F, _E, _B), jnp.float32),
        grid=(7, 2),
        in_specs=[
            pl.BlockSpec(
                (_B // 2, 128),
                lambda m, h: (jnp.minimum(2 * m + h, 7), 0),
            ),
            pl.BlockSpec(
                (_B // 2, 128),
                lambda m, h: (jnp.clip(2 * (m - 4) + h, 0, 5), 0),
            ),
        ],
        out_specs=pl.BlockSpec((4, _E, _B // 2), lambda m, h: (m, 0, h)),
    )(gp1, gp2)


def kernel(x, table):
    NW = 32
    tT = table.T
    xf = x.T.reshape(_NCHUNK, _CH)

    L1 = _untangle(tT, 0, _SPLIT_BLK)
    L2 = _untangle(tT, _SEG2_BLK, _NBLK - _SEG2_BLK)
    L1r = L1.reshape(_SPLIT_BLK * _W, _E)
    L2r = L2.reshape(_VP - _ROW_OFF, _E)

    n1 = _SPLIT_F * _CPF
    emb1 = _make_emb(n1 // NW, 0, 0, 0, _SPLIT_F // 4)
    emb2 = _make_emb((_NCHUNK - n1) // NW, n1, _SPLIT_F, _ROW_OFF,
                     (_F - _SPLIT_F + 3) // 4)

    Gp1 = emb1(xf, L1r)
    Gp2 = emb2(xf, L2r)
    Y = _repack(Gp1, Gp2)
    return jnp.transpose(Y, (2, 0, 1))


# speedup vs baseline: 1.0285x; 1.0045x over previous
import functools

import jax
import jax.numpy as jnp
from jax import lax
from jax.experimental import pallas as pl
from jax.experimental.pallas import tpu as pltpu
from jax.experimental.pallas import tpu_sc as plsc

_F = 26
_E = 32
_V = 2600000
_B = 16384
_TOTAL = _B * _F
_W = 65536
_Q = _W // 4
_NBLK = (_V + _W - 1) // _W
_VP = _NBLK * _W
_CH = 128
_NCHUNK = _TOTAL // _CH
_CPF = _B // _CH

_SPLIT_F = 20
_SPLIT_BLK = (_SPLIT_F * (_V // _F) + _W - 1) // _W
_SEG2_BLK = _SPLIT_F * (_V // _F) // _W
_ROW_OFF = _SEG2_BLK * _W


def _untangle(tT, j0, nblk):
    def body(i0, i1, i2, i3, out_ref):
        stk = jnp.concatenate([i0[...], i1[...], i2[...], i3[...]], axis=0)
        out_ref[...] = jnp.transpose(stk)

    return pl.pallas_call(
        body,
        out_shape=jax.ShapeDtypeStruct((nblk * _Q, 128), jnp.float32),
        grid=(nblk,),
        in_specs=[
            pl.BlockSpec(
                (_E, _Q),
                (lambda j, u=u: (0, jnp.minimum(4 * (j0 + j) + u, _V // _Q))),
            )
            for u in range(4)
        ],
        out_specs=pl.BlockSpec((_Q, 128), lambda j: (j, 0)),
    )(tT, tT, tT, tT)


def _make_emb(nchunks_w, base_chunk, base_f, row_off, ngroups):
    mesh = plsc.VectorSubcoreMesh(core_axis_name="c", subcore_axis_name="s")
    K = 8
    NB = nchunks_w // K

    @functools.partial(
        pl.kernel,
        mesh=mesh,
        out_type=jax.ShapeDtypeStruct((ngroups * _B, 128), jnp.float32),
        compiler_params=pltpu.CompilerParams(use_tc_tiling_on_sc=False),
        scratch_types=[
            pltpu.VMEM((nchunks_w, _CH), jnp.int32),
            pltpu.VMEM((2, K * _CH, _E), jnp.float32),
            pltpu.SemaphoreType.DMA,
            pltpu.SemaphoreType.DMA,
        ],
    )
    def _emb(x_hbm, tab_hbm, out_hbm, idxv, rows2, semg, semw):
        info = plsc.get_sparse_core_info()
        NC = info.num_cores
        wid = lax.axis_index("s") * NC + lax.axis_index("c")
        gbase = base_chunk + wid * nchunks_w
        pltpu.sync_copy(x_hbm.at[pl.ds(gbase, nchunks_w)], idxv)

        def adjust(g, carry):
            off = lax.div(gbase + g, _CPF) * (_V // _F)
            for k in range(_CH // 16):
                sl = pl.ds(k * 16, 16)
                r = idxv[g, sl] + off
                hi = lax.bitwise_and(r, jnp.int32(~(_W - 1)))
                inb = lax.bitwise_and(r, jnp.int32(_W - 1))
                u = lax.shift_right_logical(inb, 14)
                q = lax.bitwise_and(inb, jnp.int32(_Q - 1))
                s = lax.bitwise_or(
                    lax.bitwise_or(hi, lax.shift_left(q, 2)), u)
                idxv[g, sl] = s - row_off
            return carry

        lax.fori_loop(0, nchunks_w, adjust, 0)

        def _wb_window(g):
            R = gbase + g
            fl = lax.div(R, _CPF) - base_f
            m = lax.div(fl, 4)
            v = lax.rem(fl, 4)
            b0 = lax.rem(R, _CPF) * _CH
            return out_hbm.at[pl.ds(m * _B + b0, _CH), pl.ds(32 * v, 32)]

        def batch(n, carry):
            buf = lax.rem(n, 2)

            @pl.when(n >= 2)
            def _drain_prev():
                for _ in range(K):
                    pltpu.make_async_copy(
                        rows2.at[0, pl.ds(0, _CH)], _wb_window(0), semw
                    ).wait()

            cps = [
                pltpu.async_copy(
                    tab_hbm.at[idxv.at[n * K + j]],
                    rows2.at[buf, pl.ds(j * _CH, _CH)],
                    semg,
                )
                for j in range(K)
            ]
            for c in cps:
                c.wait()
            for j in range(K):
                pltpu.async_copy(
                    rows2.at[buf, pl.ds(j * _CH, _CH)],
                    _wb_window(n * K + j),
                    semw,
                )
            return carry

        lax.fori_loop(0, NB, batch, 0)
        for _ in range(2 * K):
            pltpu.make_async_copy(
                rows2.at[0, pl.ds(0, _CH)], _wb_window(0), semw
            ).wait()

    return _emb


def _repack(gp1, gp2):
    def body(i1, i2, out_ref):
        m = pl.program_id(0)
        src = jnp.where(m <= 4, i1[...], i2[...])
        out_ref[...] = jnp.transpose(src).reshape(4, _E, _B // 2)

    return pl.pallas_call(
        body,
        out_shape=jax.ShapeDtypeStruct((_F, _E, _B), jnp.float32),
        grid=(7, 2),
        in_specs=[
            pl.BlockSpec(
                (_B // 2, 128),
                lambda m, h: (jnp.minimum(2 * m + h, 9), 0),
            ),
            pl.BlockSpec(
                (_B // 2, 128),
                lambda m, h: (jnp.clip(2 * (m - 5) + h, 0, 3), 0),
            ),
        ],
        out_specs=pl.BlockSpec((4, _E, _B // 2), lambda m, h: (m, 0, h)),
    )(gp1, gp2)


def kernel(x, table):
    NW = 32
    tT = table.T
    xf = x.T.reshape(_NCHUNK, _CH)

    L1 = _untangle(tT, 0, _SPLIT_BLK)
    L2 = _untangle(tT, _SEG2_BLK, _NBLK - _SEG2_BLK)
    L1r = L1.reshape(_SPLIT_BLK * _W, _E)
    L2r = L2.reshape(_VP - _ROW_OFF, _E)

    n1 = _SPLIT_F * _CPF
    emb1 = _make_emb(n1 // NW, 0, 0, 0, _SPLIT_F // 4)
    emb2 = _make_emb((_NCHUNK - n1) // NW, n1, _SPLIT_F, _ROW_OFF,
                     (_F - _SPLIT_F + 3) // 4)

    Gp1 = emb1(xf, L1r)
    Gp2 = emb2(xf, L2r)
    Y = _repack(Gp1, Gp2)
    return jnp.transpose(Y, (2, 0, 1))
